# Initial kernel scaffold; baseline (speedup 1.0000x reference)
import jax
import jax.numpy as jnp
from jax.experimental import pallas as pl


def kernel(x, k):
    raise NotImplementedError("write your pallas kernel here")


# speedup vs baseline: 5.5249x; 5.5249x over previous
import jax
import jax.numpy as jnp
from jax.experimental import pallas as pl

N = 8192
D = 64
KOUT = 16
RB = 256


def _knn_body(x_ref, out_ref):
    i = pl.program_id(0)
    xall = x_ref[...]
    sqall = jnp.sum(xall * xall, axis=1)
    xr = x_ref[pl.ds(i * RB, RB), :]
    sqr = jnp.sum(xr * xr, axis=1)
    dot = jax.lax.dot_general(
        xr, xall, (((1,), (1,)), ((), ())),
        preferred_element_type=jnp.float32)
    d2 = sqr[:, None] + sqall[None, :] - 2.0 * dot
    iota = jax.lax.broadcasted_iota(jnp.int32, (RB, N), 1)
    big_i = jnp.int32(N)
    cols = []
    for kk in range(KOUT + 1):
        m = jnp.min(d2, axis=1)
        am = jnp.min(jnp.where(d2 == m[:, None], iota, big_i), axis=1)
        if kk > 0:
            cols.append(am)
        d2 = jnp.where(iota == am[:, None], jnp.inf, d2)
    lane = jax.lax.broadcasted_iota(jnp.int32, (RB, KOUT), 1)
    out = jnp.zeros((RB, KOUT), jnp.int32)
    for kk, am in enumerate(cols):
        out = jnp.where(lane == kk, am[:, None], out)
    out_ref[...] = out


def kernel(x, k):
    idx = pl.pallas_call(
        _knn_body,
        grid=(N // RB,),
        in_specs=[pl.BlockSpec((N, D), lambda i: (0, 0))],
        out_specs=pl.BlockSpec((RB, KOUT), lambda i: (i, 0)),
        out_shape=jax.ShapeDtypeStruct((N, KOUT), jnp.int32),
    )(x)
    return idx + jnp.asarray(k - KOUT, dtype=idx.dtype)


# speedup vs baseline: 21.2352x; 3.8435x over previous
import functools

import jax
import jax.numpy as jnp
from jax import lax
from jax.experimental import pallas as pl
from jax.experimental.pallas import tpu as pltpu
from jax.experimental.pallas import tpu_sc as plsc

N = 8192
D = 64
KOUT = 16
RB = 256
NSUPER = 4
NGROUP = 512
NWORK = 32
RPW = N // NWORK
CLAMP = 1e-12
INTMAX = 0x7FFFFFFF


def _tc_body(x_ref, xt_ref, sq_ref, dist_ref, gma_ref):
    i = pl.program_id(0)
    sqall = sq_ref[0, :]
    xr = x_ref[pl.ds(i * RB, RB), :]
    sqr = sq_ref[0, pl.ds(i * RB, RB)]
    dot = lax.dot_general(
        xr, xt_ref[...], (((1,), (0,)), ((), ())),
        preferred_element_type=jnp.float32)
    d2 = sqr[:, None] + sqall[None, :] - 2.0 * dot
    dist = jnp.sqrt(jnp.maximum(d2, jnp.float32(CLAMP)))
    dist_ref[...] = dist
    row_g = i * RB + lax.broadcasted_iota(jnp.int32, (RB, 128), 0)
    col_l = lax.broadcasted_iota(jnp.int32, (RB, 128), 1)
    gms, gas = [], []
    for t in range(NSUPER):
        m = None
        for mm in range(16):
            base = t * 2048 + mm * 128
            sl = dist[:, base:base + 128]
            colg = col_l + base
            sl = jnp.where(colg == row_g, jnp.inf, sl)
            if m is None:
                m, a = sl, colg
            else:
                upd = sl < m
                m = jnp.where(upd, sl, m)
                a = jnp.where(upd, colg, a)
        gms.append(m)
        gas.append(a)
    gma = jnp.concatenate(
        gms + [lax.bitcast_convert_type(a, jnp.float32) for a in gas], axis=1)
    gma_ref[...] = gma


def _merge16(a, b):
    ak, av = a
    bk, bv = b
    bkr = lax.rev(bk, (0,))
    bvr = lax.rev(bv, (0,))
    m = ak <= bkr
    nk = jnp.where(m, ak, bkr)
    nv = jnp.where(m, av, bvr)
    return plsc.sort_key_val(nk, nv)


def _topk16_tree(kvs):
    lvl = [plsc.sort_key_val(ck, cv) for ck, cv in kvs]
    while len(lvl) > 1:
        nxt = [_merge16(lvl[2 * j], lvl[2 * j + 1]) for j in range(len(lvl) // 2)]
        if len(lvl) % 2:
            nxt.append(lvl[-1])
        lvl = nxt
    return lvl[0]


def _fix_ties(fk, fv, iota, tkb, tvb):
    nxt = jnp.minimum(iota + 1, 15)
    prv = jnp.maximum(iota - 1, 0)
    kn = plsc.load_gather(tkb, [nxt])
    kp = plsc.load_gather(tkb, [prv])
    for parity in (0, 1):
        tvb[...] = fv
        vn = plsc.load_gather(tvb, [nxt])
        vp = plsc.load_gather(tvb, [prv])
        is_lo = (iota & 1) == parity
        swap_n = is_lo & (iota < 15) & (fk == kn) & (fv > vn)
        swap_p = (~is_lo) & (iota > 0) & (kp == fk) & (vp > fv)
        fv = jnp.where(swap_n, vn, jnp.where(swap_p, vp, fv))
    return fv


def _sc_body(dist_hbm, gma_hbm, out_hbm,
             db0, db1, gb0, gb1, outb, tkb, tvb, teb, sd0, sd1, sg0, sg1):
    nc = 2
    wid = lax.axis_index("s") * nc + lax.axis_index("c")
    base_row = wid * RPW
    iota = lax.iota(jnp.int32, 16)
    lane15 = jnp.minimum(iota + 15, 15)
    bufs = ((db0, gb0, sd0, sg0), (db1, gb1, sd1, sg1))

    def start(b, r):
        db, gb, sd, sg = bufs[b]
        pltpu.make_async_copy(dist_hbm.at[r], db, sd).start()
        pltpu.make_async_copy(gma_hbm.at[r], gb, sg).start()

    def wait(b, r):
        db, gb, sd, sg = bufs[b]
        pltpu.make_async_copy(dist_hbm.at[r], db, sd).wait()
        pltpu.make_async_copy(gma_hbm.at[r], gb, sg).wait()

    def compute(b, r_local):
        db, gb, _, _ = bufs[b]
        row_global = base_row + r_local

        kvs = [(gb[pl.ds(j * 16, 16)],
                plsc.bitcast(gb[pl.ds(NGROUP + j * 16, 16)], jnp.int32))
               for j in range(NGROUP // 16)]
        gk, gv = _topk16_tree(kvs)
        tkb[...] = gk
        taug = plsc.load_gather(tkb, [lane15])
        cg = jnp.sum((gk < taug).astype(jnp.int32))
        ekvs = []
        for j in range(NGROUP // 16):
            gmj = gb[pl.ds(j * 16, 16)]
            gaj = plsc.bitcast(gb[pl.ds(NGROUP + j * 16, 16)], jnp.int32)
            e = jnp.where(gmj == taug, gaj, jnp.int32(INTMAX))
            ekvs.append((e, e))
        eqg, _ = _topk16_tree(ekvs)
        teb[...] = eqg
        eqsh = plsc.load_gather(teb, [jnp.maximum(iota - cg, 0)])
        gcols = jnp.where(iota < cg, gv, eqsh)

        basev = ((gcols >> 11) << 11) | (gcols & 127)
        ckvs = []
        for mm in range(16):
            idxv = basev + jnp.int32(mm * 128)
            ck = plsc.load_gather(db, [idxv])
            ck = jnp.where(idxv == row_global, jnp.inf, ck)
            ckvs.append((ck, idxv))
        fk, fv = _topk16_tree(ckvs)
        tkb[...] = fk
        tau = plsc.load_gather(tkb, [lane15])
        c = jnp.sum((fk < tau).astype(jnp.int32))
        fv = _fix_ties(fk, fv, iota, tkb, tvb)
        eq = []
        for mm in range(16):
            idxv = basev + jnp.int32(mm * 128)
            ck = plsc.load_gather(db, [idxv])
            ck = jnp.where(idxv == row_global, jnp.inf, ck)
            e = jnp.where(ck == tau, idxv, jnp.int32(INTMAX))
            eq.append((e, e))
        eqk, _ = _topk16_tree(eq)
        teb[...] = eqk
        eqshc = plsc.load_gather(teb, [jnp.maximum(iota - c, 0)])
        outb[r_local, :] = jnp.where(iota < c, fv, eqshc)

    start(0, base_row)

    def body(i2, carry):
        for b in range(2):
            r = 2 * i2 + b

            @pl.when(r + 1 < RPW)
            def _():
                start(1 - b, base_row + r + 1)

            wait(b, base_row + r)
            compute(b, r)
        return carry

    lax.fori_loop(0, RPW // 2, body, jnp.int32(0))
    pltpu.sync_copy(outb, out_hbm.at[pl.ds(base_row, RPW)])


_sc_call = functools.partial(
    pl.kernel,
    out_type=jax.ShapeDtypeStruct((N, KOUT), jnp.int32),
    mesh=plsc.VectorSubcoreMesh(core_axis_name="c", subcore_axis_name="s"),
    compiler_params=pltpu.CompilerParams(needs_layout_passes=False),
    scratch_types=[
        pltpu.VMEM((N,), jnp.float32),
        pltpu.VMEM((N,), jnp.float32),
        pltpu.VMEM((2 * NGROUP,), jnp.float32),
        pltpu.VMEM((2 * NGROUP,), jnp.float32),
        pltpu.VMEM((RPW, KOUT), jnp.int32),
        pltpu.VMEM((16,), jnp.float32),
        pltpu.VMEM((16,), jnp.int32),
        pltpu.VMEM((16,), jnp.int32),
        pltpu.SemaphoreType.DMA,
        pltpu.SemaphoreType.DMA,
        pltpu.SemaphoreType.DMA,
        pltpu.SemaphoreType.DMA,
    ],
)(_sc_body)


def kernel(x, k):
    sq = jnp.sum(x * x, axis=1)
    dist, gma = pl.pallas_call(
        _tc_body,
        grid=(N // RB,),
        in_specs=[
            pl.BlockSpec((N, D), lambda i: (0, 0)),
            pl.BlockSpec((D, N), lambda i: (0, 0)),
            pl.BlockSpec((1, N), lambda i: (0, 0)),
        ],
        out_specs=[
            pl.BlockSpec((RB, N), lambda i: (i, 0)),
            pl.BlockSpec((RB, 2 * NGROUP), lambda i: (i, 0)),
        ],
        out_shape=[
            jax.ShapeDtypeStruct((N, N), jnp.float32),
            jax.ShapeDtypeStruct((N, 2 * NGROUP), jnp.float32),
        ],
    )(x, x.T, sq[None, :])
    idx = _sc_call(dist, gma)
    return idx + jnp.asarray(k - KOUT, dtype=idx.dtype)


# speedup vs baseline: 24.7120x; 1.1637x over previous
import functools

import jax
import jax.numpy as jnp
from jax import lax
from jax.experimental import pallas as pl
from jax.experimental.pallas import tpu as pltpu
from jax.experimental.pallas import tpu_sc as plsc

N = 8192
D = 64
KOUT = 16
RB = 256
NSUPER = 4
NGROUP = 512
NWORK = 32
RPW = N // NWORK
CLAMP = 1e-12
INTMAX = 0x7FFFFFFF


NH = N // 2
RPW2 = NH // NWORK


def _tc_body(off, x_ref, xt_ref, sq_ref, dist_ref, gma_ref):
    i = pl.program_id(0)
    sqall = sq_ref[0, :]
    xr = x_ref[pl.ds(off + i * RB, RB), :]
    sqr = sq_ref[0, pl.ds(off + i * RB, RB)]
    dot = lax.dot_general(
        xr, xt_ref[...], (((1,), (0,)), ((), ())),
        preferred_element_type=jnp.float32)
    d2 = sqr[:, None] + sqall[None, :] - 2.0 * dot
    dist = jnp.sqrt(jnp.maximum(d2, jnp.float32(CLAMP)))
    dist_ref[...] = dist
    row_g = off + i * RB + lax.broadcasted_iota(jnp.int32, (RB, 128), 0)
    col_l = lax.broadcasted_iota(jnp.int32, (RB, 128), 1)
    gms, gas = [], []
    for t in range(NSUPER):
        m = None
        for mm in range(16):
            base = t * 2048 + mm * 128
            sl = dist[:, base:base + 128]
            colg = col_l + base
            sl = jnp.where(colg == row_g, jnp.inf, sl)
            if m is None:
                m, a = sl, colg
            else:
                upd = sl < m
                m = jnp.where(upd, sl, m)
                a = jnp.where(upd, colg, a)
        gms.append(m)
        gas.append(a)
    gma = jnp.concatenate(
        gms + [lax.bitcast_convert_type(a, jnp.float32) for a in gas], axis=1)
    gma_ref[...] = gma


def _merge16(a, b):
    ak, av = a
    bk, bv = b
    bkr = lax.rev(bk, (0,))
    bvr = lax.rev(bv, (0,))
    m = ak <= bkr
    nk = jnp.where(m, ak, bkr)
    nv = jnp.where(m, av, bvr)
    return plsc.sort_key_val(nk, nv)


def _topk16_tree(kvs):
    lvl = [plsc.sort_key_val(ck, cv) for ck, cv in kvs]
    while len(lvl) > 1:
        nxt = [_merge16(lvl[2 * j], lvl[2 * j + 1]) for j in range(len(lvl) // 2)]
        if len(lvl) % 2:
            nxt.append(lvl[-1])
        lvl = nxt
    return lvl[0]


def _fix_ties(fk, fv, iota, tkb, tvb):
    nxt = jnp.minimum(iota + 1, 15)
    prv = jnp.maximum(iota - 1, 0)
    kn = plsc.load_gather(tkb, [nxt])
    kp = plsc.load_gather(tkb, [prv])
    for parity in (0, 1):
        tvb[...] = fv
        vn = plsc.load_gather(tvb, [nxt])
        vp = plsc.load_gather(tvb, [prv])
        is_lo = (iota & 1) == parity
        swap_n = is_lo & (iota < 15) & (fk == kn) & (fv > vn)
        swap_p = (~is_lo) & (iota > 0) & (kp == fk) & (vp > fv)
        fv = jnp.where(swap_n, vn, jnp.where(swap_p, vp, fv))
    return fv


def _sc_body(off, dist_hbm, gma_hbm, out_hbm,
             db0, db1, gb0, gb1, outb, tkb, tvb, teb, sd0, sd1, sg0, sg1):
    nc = 2
    wid = lax.axis_index("s") * nc + lax.axis_index("c")
    base_row = wid * RPW2
    iota = lax.iota(jnp.int32, 16)
    lane15 = jnp.minimum(iota + 15, 15)
    bufs = ((db0, gb0, sd0, sg0), (db1, gb1, sd1, sg1))

    def start(b, r):
        db, gb, sd, sg = bufs[b]
        pltpu.make_async_copy(dist_hbm.at[r], db, sd).start()
        pltpu.make_async_copy(gma_hbm.at[r], gb, sg).start()

    def wait(b, r):
        db, gb, sd, sg = bufs[b]
        pltpu.make_async_copy(dist_hbm.at[r], db, sd).wait()
        pltpu.make_async_copy(gma_hbm.at[r], gb, sg).wait()

    def compute(b, r_local):
        db, gb, _, _ = bufs[b]
        row_global = off + base_row + r_local

        kvs = [(gb[pl.ds(j * 16, 16)],
                plsc.bitcast(gb[pl.ds(NGROUP + j * 16, 16)], jnp.int32))
               for j in range(NGROUP // 16)]
        gk, gv = _topk16_tree(kvs)
        tkb[...] = gk
        taug = plsc.load_gather(tkb, [lane15])
        cg = jnp.sum((gk < taug).astype(jnp.int32))
        ekvs = []
        for j in range(NGROUP // 16):
            gmj = gb[pl.ds(j * 16, 16)]
            gaj = plsc.bitcast(gb[pl.ds(NGROUP + j * 16, 16)], jnp.int32)
            e = jnp.where(gmj == taug, gaj, jnp.int32(INTMAX))
            ekvs.append((e, e))
        eqg, _ = _topk16_tree(ekvs)
        teb[...] = eqg
        eqsh = plsc.load_gather(teb, [jnp.maximum(iota - cg, 0)])
        gcols = jnp.where(iota < cg, gv, eqsh)

        basev = ((gcols >> 11) << 11) | (gcols & 127)
        ckvs = []
        for mm in range(16):
            idxv = basev + jnp.int32(mm * 128)
            ck = plsc.load_gather(db, [idxv])
            ck = jnp.where(idxv == row_global, jnp.inf, ck)
            ckvs.append((ck, idxv))
        fk, fv = _topk16_tree(ckvs)
        tkb[...] = fk
        tau = plsc.load_gather(tkb, [lane15])
        c = jnp.sum((fk < tau).astype(jnp.int32))
        fv = _fix_ties(fk, fv, iota, tkb, tvb)
        eq = []
        for mm in range(16):
            idxv = basev + jnp.int32(mm * 128)
            ck = plsc.load_gather(db, [idxv])
            ck = jnp.where(idxv == row_global, jnp.inf, ck)
            e = jnp.where(ck == tau, idxv, jnp.int32(INTMAX))
            eq.append((e, e))
        eqk, _ = _topk16_tree(eq)
        teb[...] = eqk
        eqshc = plsc.load_gather(teb, [jnp.maximum(iota - c, 0)])
        outb[r_local, :] = jnp.where(iota < c, fv, eqshc)

    start(0, base_row)

    def body(i2, carry):
        for b in range(2):
            r = 2 * i2 + b

            @pl.when(r + 1 < RPW2)
            def _():
                start(1 - b, base_row + r + 1)

            wait(b, base_row + r)
            compute(b, r)
        return carry

    lax.fori_loop(0, RPW2 // 2, body, jnp.int32(0))
    pltpu.sync_copy(outb, out_hbm.at[pl.ds(base_row, RPW2)])


def _make_sc(off):
    return functools.partial(
        pl.kernel,
        out_type=jax.ShapeDtypeStruct((NH, KOUT), jnp.int32),
        mesh=plsc.VectorSubcoreMesh(core_axis_name="c", subcore_axis_name="s"),
        compiler_params=pltpu.CompilerParams(needs_layout_passes=False),
        scratch_types=[
            pltpu.VMEM((N,), jnp.float32),
            pltpu.VMEM((N,), jnp.float32),
            pltpu.VMEM((2 * NGROUP,), jnp.float32),
            pltpu.VMEM((2 * NGROUP,), jnp.float32),
            pltpu.VMEM((RPW2, KOUT), jnp.int32),
            pltpu.VMEM((16,), jnp.float32),
            pltpu.VMEM((16,), jnp.int32),
            pltpu.VMEM((16,), jnp.int32),
            pltpu.SemaphoreType.DMA,
            pltpu.SemaphoreType.DMA,
            pltpu.SemaphoreType.DMA,
            pltpu.SemaphoreType.DMA,
        ],
    )(functools.partial(_sc_body, off))


def _make_tc(off):
    return pl.pallas_call(
        functools.partial(_tc_body, off),
        grid=(NH // RB,),
        in_specs=[
            pl.BlockSpec((N, D), lambda i: (0, 0)),
            pl.BlockSpec((D, N), lambda i: (0, 0)),
            pl.BlockSpec((1, N), lambda i: (0, 0)),
        ],
        out_specs=[
            pl.BlockSpec((RB, N), lambda i: (i, 0)),
            pl.BlockSpec((RB, 2 * NGROUP), lambda i: (i, 0)),
        ],
        out_shape=[
            jax.ShapeDtypeStruct((NH, N), jnp.float32),
            jax.ShapeDtypeStruct((NH, 2 * NGROUP), jnp.float32),
        ],
    )


_TC_CALLS = [_make_tc(0), _make_tc(NH)]
_SC_CALLS = [_make_sc(0), _make_sc(NH)]


def kernel(x, k):
    sq = jnp.sum(x * x, axis=1)[None, :]
    xt = x.T
    halves = []
    for h in range(2):
        dist, gma = _TC_CALLS[h](x, xt, sq)
        halves.append(_SC_CALLS[h](dist, gma))
    idx = jnp.concatenate(halves, axis=0)
    return idx + jnp.asarray(k - KOUT, dtype=idx.dtype)


# speedup vs baseline: 25.8383x; 1.0456x over previous
import functools

import jax
import jax.numpy as jnp
from jax import lax
from jax.experimental import pallas as pl
from jax.experimental.pallas import tpu as pltpu
from jax.experimental.pallas import tpu_sc as plsc

N = 8192
D = 64
KOUT = 16
RB = 256
NSUPER = 4
NGROUP = 512
NWORK = 32
RPW = N // NWORK
CLAMP = 1e-12
INTMAX = 0x7FFFFFFF


NSPLIT = 4
NH = N // NSPLIT

RPW2 = NH // NWORK


def _tc_body(off, x_ref, xt_ref, sq_ref, dist_ref, gma_ref):
    i = pl.program_id(0)
    sqall = sq_ref[0, :]
    xr = x_ref[pl.ds(off + i * RB, RB), :]
    sqr = sq_ref[0, pl.ds(off + i * RB, RB)]
    dot = lax.dot_general(
        xr, xt_ref[...], (((1,), (0,)), ((), ())),
        preferred_element_type=jnp.float32)
    d2 = sqr[:, None] + sqall[None, :] - 2.0 * dot
    dist = jnp.sqrt(jnp.maximum(d2, jnp.float32(CLAMP)))
    dist_ref[...] = dist
    row_g = off + i * RB + lax.broadcasted_iota(jnp.int32, (RB, 128), 0)
    col_l = lax.broadcasted_iota(jnp.int32, (RB, 128), 1)
    gms, gas = [], []
    for t in range(NSUPER):
        m = None
        for mm in range(16):
            base = t * 2048 + mm * 128
            sl = dist[:, base:base + 128]
            colg = col_l + base
            sl = jnp.where(colg == row_g, jnp.inf, sl)
            if m is None:
                m, a = sl, colg
            else:
                upd = sl < m
                m = jnp.where(upd, sl, m)
                a = jnp.where(upd, colg, a)
        gms.append(m)
        gas.append(a)
    gma = jnp.concatenate(
        gms + [lax.bitcast_convert_type(a, jnp.float32) for a in gas], axis=1)
    gma_ref[...] = gma


def _merge16(a, b):
    ak, av = a
    bk, bv = b
    bkr = lax.rev(bk, (0,))
    bvr = lax.rev(bv, (0,))
    m = ak <= bkr
    nk = jnp.where(m, ak, bkr)
    nv = jnp.where(m, av, bvr)
    return plsc.sort_key_val(nk, nv)


def _topk16_tree(kvs):
    lvl = [plsc.sort_key_val(ck, cv) for ck, cv in kvs]
    while len(lvl) > 1:
        nxt = [_merge16(lvl[2 * j], lvl[2 * j + 1]) for j in range(len(lvl) // 2)]
        if len(lvl) % 2:
            nxt.append(lvl[-1])
        lvl = nxt
    return lvl[0]


def _fix_ties(fk, fv, iota, tkb, tvb):
    nxt = jnp.minimum(iota + 1, 15)
    prv = jnp.maximum(iota - 1, 0)
    kn = plsc.load_gather(tkb, [nxt])
    kp = plsc.load_gather(tkb, [prv])
    for parity in (0, 1):
        tvb[...] = fv
        vn = plsc.load_gather(tvb, [nxt])
        vp = plsc.load_gather(tvb, [prv])
        is_lo = (iota & 1) == parity
        swap_n = is_lo & (iota < 15) & (fk == kn) & (fv > vn)
        swap_p = (~is_lo) & (iota > 0) & (kp == fk) & (vp > fv)
        fv = jnp.where(swap_n, vn, jnp.where(swap_p, vp, fv))
    return fv


def _sc_body(off, dist_hbm, gma_hbm, out_hbm,
             db0, db1, gb0, gb1, outb, tkb, tvb, teb, sd0, sd1, sg0, sg1):
    nc = 2
    wid = lax.axis_index("s") * nc + lax.axis_index("c")
    base_row = wid * RPW2
    iota = lax.iota(jnp.int32, 16)
    lane15 = jnp.minimum(iota + 15, 15)
    bufs = ((db0, gb0, sd0, sg0), (db1, gb1, sd1, sg1))

    def start(b, r):
        db, gb, sd, sg = bufs[b]
        pltpu.make_async_copy(dist_hbm.at[r], db, sd).start()
        pltpu.make_async_copy(gma_hbm.at[r], gb, sg).start()

    def wait(b, r):
        db, gb, sd, sg = bufs[b]
        pltpu.make_async_copy(dist_hbm.at[r], db, sd).wait()
        pltpu.make_async_copy(gma_hbm.at[r], gb, sg).wait()

    def compute(b, r_local):
        db, gb, _, _ = bufs[b]
        row_global = off + base_row + r_local

        kvs = [(gb[pl.ds(j * 16, 16)],
                plsc.bitcast(gb[pl.ds(NGROUP + j * 16, 16)], jnp.int32))
               for j in range(NGROUP // 16)]
        gk, gv = _topk16_tree(kvs)
        tkb[...] = gk
        taug = plsc.load_gather(tkb, [lane15])
        cg = jnp.sum((gk < taug).astype(jnp.int32))
        ekvs = []
        for j in range(NGROUP // 16):
            gmj = gb[pl.ds(j * 16, 16)]
            gaj = plsc.bitcast(gb[pl.ds(NGROUP + j * 16, 16)], jnp.int32)
            e = jnp.where(gmj == taug, gaj, jnp.int32(INTMAX))
            ekvs.append((e, e))
        eqg, _ = _topk16_tree(ekvs)
        teb[...] = eqg
        eqsh = plsc.load_gather(teb, [jnp.maximum(iota - cg, 0)])
        gcols = jnp.where(iota < cg, gv, eqsh)

        basev = ((gcols >> 11) << 11) | (gcols & 127)
        ckvs = []
        for mm in range(16):
            idxv = basev + jnp.int32(mm * 128)
            ck = plsc.load_gather(db, [idxv])
            ck = jnp.where(idxv == row_global, jnp.inf, ck)
            ckvs.append((ck, idxv))
        fk, fv = _topk16_tree(ckvs)
        tkb[...] = fk
        tau = plsc.load_gather(tkb, [lane15])
        c = jnp.sum((fk < tau).astype(jnp.int32))
        fv = _fix_ties(fk, fv, iota, tkb, tvb)
        eq = []
        for mm in range(16):
            idxv = basev + jnp.int32(mm * 128)
            ck = plsc.load_gather(db, [idxv])
            ck = jnp.where(idxv == row_global, jnp.inf, ck)
            e = jnp.where(ck == tau, idxv, jnp.int32(INTMAX))
            eq.append((e, e))
        eqk, _ = _topk16_tree(eq)
        teb[...] = eqk
        eqshc = plsc.load_gather(teb, [jnp.maximum(iota - c, 0)])
        outb[r_local, :] = jnp.where(iota < c, fv, eqshc)

    start(0, base_row)

    def body(i2, carry):
        for b in range(2):
            r = 2 * i2 + b

            @pl.when(r + 1 < RPW2)
            def _():
                start(1 - b, base_row + r + 1)

            wait(b, base_row + r)
            compute(b, r)
        return carry

    lax.fori_loop(0, RPW2 // 2, body, jnp.int32(0))
    pltpu.sync_copy(outb, out_hbm.at[pl.ds(base_row, RPW2)])


def _make_sc(off):
    return functools.partial(
        pl.kernel,
        out_type=jax.ShapeDtypeStruct((NH, KOUT), jnp.int32),
        mesh=plsc.VectorSubcoreMesh(core_axis_name="c", subcore_axis_name="s"),
        compiler_params=pltpu.CompilerParams(needs_layout_passes=False),
        scratch_types=[
            pltpu.VMEM((N,), jnp.float32),
            pltpu.VMEM((N,), jnp.float32),
            pltpu.VMEM((2 * NGROUP,), jnp.float32),
            pltpu.VMEM((2 * NGROUP,), jnp.float32),
            pltpu.VMEM((RPW2, KOUT), jnp.int32),
            pltpu.VMEM((16,), jnp.float32),
            pltpu.VMEM((16,), jnp.int32),
            pltpu.VMEM((16,), jnp.int32),
            pltpu.SemaphoreType.DMA,
            pltpu.SemaphoreType.DMA,
            pltpu.SemaphoreType.DMA,
            pltpu.SemaphoreType.DMA,
        ],
    )(functools.partial(_sc_body, off))


def _make_tc(off):
    return pl.pallas_call(
        functools.partial(_tc_body, off),
        grid=(NH // RB,),
        in_specs=[
            pl.BlockSpec((N, D), lambda i: (0, 0)),
            pl.BlockSpec((D, N), lambda i: (0, 0)),
            pl.BlockSpec((1, N), lambda i: (0, 0)),
        ],
        out_specs=[
            pl.BlockSpec((RB, N), lambda i: (i, 0)),
            pl.BlockSpec((RB, 2 * NGROUP), lambda i: (i, 0)),
        ],
        out_shape=[
            jax.ShapeDtypeStruct((NH, N), jnp.float32),
            jax.ShapeDtypeStruct((NH, 2 * NGROUP), jnp.float32),
        ],
    )


_TC_CALLS = [_make_tc(h * NH) for h in range(NSPLIT)]
_SC_CALLS = [_make_sc(h * NH) for h in range(NSPLIT)]


def kernel(x, k):
    sq = jnp.sum(x * x, axis=1)[None, :]
    xt = x.T
    chunks = []
    for h in range(NSPLIT):
        dist, gma = _TC_CALLS[h](x, xt, sq)
        chunks.append(_SC_CALLS[h](dist, gma))
    idx = jnp.concatenate(chunks, axis=0)
    return idx + jnp.asarray(k - KOUT, dtype=idx.dtype)
